# baseline (speedup 1.0000x reference)
import functools

import jax
import jax.numpy as jnp
import numpy as np
from jax import lax
from jax.experimental import pallas as pl
from jax.experimental.pallas import tpu as pltpu
from jax.experimental.pallas import tpu_sc as plsc

_NC = 2
_NS = 16
_NW = _NC * _NS
_D = 512
_N = 512
_G = 32 * 3
_GSC = 48
_GTC = _G - _GSC
_ROWS_ALL = _G * _N
_ROWS_SC = _GSC * _N
_BPW = _ROWS_SC // _NW
_C = 96
_NCH = _BPW // _C
_NP = _NCH // 2
_GB = 3
_STEPS = _GTC // _GB


def _gather_rows():
    mesh = plsc.VectorSubcoreMesh(core_axis_name="c", subcore_axis_name="s")

    @functools.partial(
        pl.kernel,
        mesh=mesh,
        out_type=jax.ShapeDtypeStruct((_ROWS_SC, _D), jnp.float32),
        scratch_types=[
            pltpu.VMEM((_NCH, _C), jnp.int32),
            pltpu.VMEM((_C, _D), jnp.float32),
            pltpu.VMEM((_C, _D), jnp.float32),
            pltpu.SemaphoreType.DMA,
            pltpu.SemaphoreType.DMA,
            pltpu.SemaphoreType.DMA,
            pltpu.SemaphoreType.DMA,
        ],
    )
    def k(tbl_hbm, idx_hbm, out_hbm, idx_v, rows0, rows1, gs0, gs1, ss0, ss1):
        wid = lax.axis_index("s") * _NC + lax.axis_index("c")
        base = wid * _BPW
        pltpu.sync_copy(idx_hbm.at[wid], idx_v)

        pltpu.async_copy(tbl_hbm.at[idx_v.at[0]], rows0, gs0)

        def body(p, carry):
            j0 = 2 * p
            j1 = j0 + 1

            @pl.when(p > 0)
            def _():
                pltpu.make_async_copy(
                    rows1, out_hbm.at[pl.ds(base, _C)], ss1).wait()

            pltpu.async_copy(tbl_hbm.at[idx_v.at[j1]], rows1, gs1)

            pltpu.make_async_copy(
                tbl_hbm.at[idx_v.at[j0]], rows0, gs0).wait()
            pltpu.async_copy(rows0, out_hbm.at[pl.ds(base + j0 * _C, _C)], ss0)

            @pl.when(p < _NP - 1)
            def _():
                pltpu.make_async_copy(
                    rows0, out_hbm.at[pl.ds(base, _C)], ss0).wait()
                pltpu.async_copy(tbl_hbm.at[idx_v.at[j0 + 2]], rows0, gs0)

            pltpu.make_async_copy(
                tbl_hbm.at[idx_v.at[j1]], rows1, gs1).wait()
            pltpu.async_copy(rows1, out_hbm.at[pl.ds(base + j1 * _C, _C)], ss1)
            return carry

        lax.fori_loop(0, _NP, body, 0)

        pltpu.make_async_copy(rows0, out_hbm.at[pl.ds(base, _C)], ss0).wait()
        pltpu.make_async_copy(rows1, out_hbm.at[pl.ds(base, _C)], ss1).wait()

    return k


_KERNEL = _gather_rows()


def _tc_body(p_ref, x_ref, o_ref):
    p = p_ref[...]
    for i in range(_GB):
        o_ref[i] = jax.lax.dot_general(
            p, x_ref[i],
            dimension_numbers=(((1,), (0,)), ((), ())),
            precision=jax.lax.Precision.DEFAULT,
            preferred_element_type=jnp.float32,
        )


_TC = pl.pallas_call(
    _tc_body,
    grid=(_STEPS,),
    in_specs=[
        pl.BlockSpec((_N, _N), lambda g: (0, 0)),
        pl.BlockSpec((_GB, _N, _D), lambda g: (g, 0, 0)),
    ],
    out_specs=pl.BlockSpec((_GB, _N, _D), lambda g: (g, 0, 0)),
    out_shape=jax.ShapeDtypeStruct((_GTC, _N, _D), jnp.float32),
)

_PERM = np.asarray(jax.random.permutation(jax.random.key(42), _N),
                   dtype=np.int32)
_GIDX = (np.arange(_GSC, dtype=np.int32)[:, None] * _N + _PERM[None, :])
_GIDX = _GIDX.reshape(_NW, _NCH, _C)
_PMAT = np.zeros((_N, _N), dtype=np.float32)
_PMAT[np.arange(_N), _PERM] = 1.0


@jax.jit
def kernel(img):
    gidx = jnp.asarray(_GIDX)
    pmat = jnp.asarray(_PMAT)
    tbl = img.reshape(_ROWS_ALL, _D)
    sc_out = _KERNEL(tbl, gidx)
    tc_out = _TC(pmat, img.reshape(_G, _N, _D)[_GSC:])
    out = jnp.concatenate(
        [sc_out.reshape(_GSC, _N, _D), tc_out], axis=0)
    return out.reshape(img.shape)


# speedup vs baseline: 2.0183x; 2.0183x over previous
import functools

import jax
import jax.numpy as jnp
import numpy as np
from jax import lax
from jax.experimental import pallas as pl
from jax.experimental.pallas import tpu as pltpu
from jax.experimental.pallas import tpu_sc as plsc

_NC = 2
_NS = 16
_NW = _NC * _NS
_D = 512
_N = 512
_G = 32 * 3
_ROWS = _G * _N
_BPW = _ROWS // _NW
_C = 96
_NCH = _BPW // _C
_NP = _NCH // 2


def _gather_rows():
    mesh = plsc.VectorSubcoreMesh(core_axis_name="c", subcore_axis_name="s")

    @functools.partial(
        pl.kernel,
        mesh=mesh,
        out_type=jax.ShapeDtypeStruct((_ROWS, _D), jnp.float32),
        scratch_types=[
            pltpu.VMEM((_NCH, _C), jnp.int32),
            pltpu.VMEM((_C, _D), jnp.float32),
            pltpu.VMEM((_C, _D), jnp.float32),
            pltpu.SemaphoreType.DMA,
            pltpu.SemaphoreType.DMA,
            pltpu.SemaphoreType.DMA,
            pltpu.SemaphoreType.DMA,
        ],
    )
    def k(tbl_hbm, idx_hbm, out_hbm, idx_v, rows0, rows1, gs0, gs1, ss0, ss1):
        wid = lax.axis_index("s") * _NC + lax.axis_index("c")
        base = wid * _BPW
        pltpu.sync_copy(idx_hbm.at[wid], idx_v)

        pltpu.async_copy(tbl_hbm.at[idx_v.at[0]], rows0, gs0)

        def body(p, carry):
            j0 = 2 * p
            j1 = j0 + 1

            @pl.when(p > 0)
            def _():
                pltpu.make_async_copy(
                    rows1, out_hbm.at[pl.ds(base, _C)], ss1).wait()

            pltpu.async_copy(tbl_hbm.at[idx_v.at[j1]], rows1, gs1)

            pltpu.make_async_copy(
                tbl_hbm.at[idx_v.at[j0]], rows0, gs0).wait()
            pltpu.async_copy(rows0, out_hbm.at[pl.ds(base + j0 * _C, _C)], ss0)

            @pl.when(p < _NP - 1)
            def _():
                pltpu.make_async_copy(
                    rows0, out_hbm.at[pl.ds(base, _C)], ss0).wait()
                pltpu.async_copy(
                    tbl_hbm.at[idx_v.at[j0 + 2]], rows0, gs0)

            pltpu.make_async_copy(
                tbl_hbm.at[idx_v.at[j1]], rows1, gs1).wait()
            pltpu.async_copy(rows1, out_hbm.at[pl.ds(base + j1 * _C, _C)], ss1)
            return carry

        lax.fori_loop(0, _NP, body, 0)

        pltpu.make_async_copy(rows0, out_hbm.at[pl.ds(base, _C)], ss0).wait()
        pltpu.make_async_copy(rows1, out_hbm.at[pl.ds(base, _C)], ss1).wait()

    return k


_KERNEL = _gather_rows()

_PERM = np.asarray([121, 480, 35, 130, 263, 148, 197, 410, 398, 45, 176, 462, 446, 366, 257, 179, 139, 315, 501, 188, 312, 499, 318, 448, 304, 99, 309, 144, 152, 189, 487, 325, 31, 112, 495, 356, 493, 507, 268, 429, 409, 85, 63, 117, 417, 174, 441, 509, 481, 272, 114, 254, 82, 65, 7, 350, 4, 101, 463, 452, 444, 102, 78, 163, 157, 302, 183, 29, 240, 177, 278, 259, 108, 305, 83, 129, 367, 212, 277, 504, 300, 44, 211, 16, 58, 123, 37, 336, 111, 19, 61, 447, 2, 142, 34, 369, 339, 156, 436, 5, 461, 415, 90, 363, 175, 167, 284, 379, 251, 110, 72, 155, 178, 323, 291, 388, 269, 354, 368, 219, 510, 153, 30, 275, 42, 186, 342, 406, 468, 439, 307, 256, 419, 246, 3, 362, 380, 327, 393, 70, 378, 400, 271, 488, 311, 67, 273, 223, 422, 39, 56, 274, 192, 169, 349, 218, 195, 476, 173, 245, 241, 69, 383, 80, 22, 6, 321, 199, 345, 118, 235, 54, 442, 479, 423, 266, 77, 425, 147, 18, 340, 298, 249, 294, 375, 382, 10, 11, 234, 53, 236, 455, 94, 332, 511, 331, 437, 353, 489, 287, 32, 217, 283, 355, 407, 159, 440, 15, 470, 184, 49, 137, 50, 138, 20, 445, 237, 280, 253, 185, 460, 43, 389, 335, 258, 370, 344, 92, 8, 503, 324, 140, 233, 24, 81, 239, 314, 453, 96, 475, 467, 154, 135, 472, 490, 469, 500, 264, 160, 106, 128, 265, 426, 386, 191, 9, 200, 40, 187, 71, 346, 438, 333, 248, 164, 207, 93, 59, 201, 158, 210, 420, 402, 75, 508, 131, 411, 97, 66, 25, 196, 424, 364, 497, 242, 338, 206, 243, 397, 341, 450, 414, 238, 295, 432, 431, 308, 73, 320, 13, 52, 491, 203, 289, 303, 202, 255, 194, 88, 250, 337, 62, 230, 150, 261, 330, 262, 209, 132, 357, 87, 76, 198, 486, 60, 244, 457, 47, 392, 374, 276, 33, 79, 451, 180, 403, 247, 14, 459, 286, 421, 458, 228, 17, 38, 86, 231, 190, 232, 482, 23, 105, 484, 395, 427, 301, 474, 376, 405, 494, 471, 391, 313, 220, 0, 473, 145, 371, 213, 226, 381, 133, 281, 41, 64, 416, 21, 443, 161, 279, 285, 166, 124, 116, 449, 26, 165, 168, 193, 57, 208, 181, 89, 146, 182, 126, 125, 297, 1, 115, 28, 113, 225, 361, 351, 465, 172, 377, 162, 48, 170, 466, 505, 227, 36, 252, 502, 492, 119, 151, 385, 306, 120, 372, 390, 224, 122, 270, 100, 418, 433, 329, 365, 396, 91, 222, 55, 496, 498, 103, 51, 293, 215, 384, 127, 98, 483, 506, 282, 107, 27, 322, 74, 136, 229, 319, 328, 430, 343, 204, 221, 296, 12, 134, 454, 477, 408, 109, 84, 428, 317, 358, 394, 299, 205, 171, 288, 143, 68, 267, 216, 435, 149, 485, 434, 141, 464, 334, 404, 104, 352, 95, 387, 316, 214, 290, 46, 310, 348, 401, 260, 478, 292, 359, 326, 347, 456, 399, 373, 412, 360, 413], dtype=np.int32)
_GIDX = (np.arange(_G, dtype=np.int32)[:, None] * _N + _PERM[None, :])
_GIDX = _GIDX.reshape(_NW, _NCH, _C)


@jax.jit
def kernel(img):
    gidx = jnp.asarray(_GIDX)
    tbl = img.reshape(_ROWS, _D)
    out = _KERNEL(tbl, gidx)
    return out.reshape(img.shape)
